# baseline (speedup 1.0000x reference)
import math

import numpy as np
import jax
import jax.numpy as jnp
from jax import lax
from jax.experimental import pallas as pl
from jax.experimental.pallas import tpu as pltpu
from jax.experimental.pallas import tpu_sc as plsc

_D = 1024
_B = 4
_S = 4096
_NC = 2
_NS = 16
_NW = _NC * _NS
_PPW = _S // _NW
_CH = 8
_NCHUNK = _PPW // _CH
_SCALE = math.sqrt(_D)
_LANES = 16
_VPR = _D // _LANES
_NGRP = 3


def _pos_encoding() -> np.ndarray:
    pos = np.arange(_S, dtype=np.float32)[:, None]
    i = np.arange(_D // 2, dtype=np.float32)[None, :]
    angle = pos / np.power(10000.0, (2.0 * i) / _D)
    pe = np.zeros((_S, _D), dtype=np.float32)
    pe[:, 0::2] = np.sin(angle)
    pe[:, 1::2] = np.cos(angle)
    return pe


_PE = _pos_encoding()


def _compute(bufs, pe_v):
    @pl.loop(0, _CH // 2)
    def _rows(r2):
        @pl.loop(0, _D // 32, unroll=2)
        def _vecs(j):
            sl0 = pl.ds(j * 32, _LANES)
            sl1 = pl.ds(j * 32 + _LANES, _LANES)
            for rr in range(2):
                ab = pe_v[r2, rr, pl.ds(j * 32, 32)]
                p0 = ab[0:16].astype(jnp.float32)
                p1 = ab[16:32].astype(jnp.float32)
                r = 2 * r2 + rr
                for buf in bufs:
                    buf[r, sl0] = buf[r, sl0] * _SCALE + p0
                    buf[r, sl1] = buf[r, sl1] * _SCALE + p1


def _body(table, seqs, pe, out, idx_v, pe0, pe1, *rest):
    grp = [[rest[g * _B + b] for b in range(_B)] for g in range(_NGRP)]
    gsems = rest[_NGRP * _B:_NGRP * _B + _NGRP]
    ssems = rest[_NGRP * _B + _NGRP:_NGRP * _B + 2 * _NGRP]
    psems = rest[_NGRP * _B + 2 * _NGRP:]
    pes = (pe0, pe1)

    wid = lax.axis_index("s") * _NC + lax.axis_index("c")
    pos0 = wid * _PPW

    for b in range(_B):
        pltpu.sync_copy(seqs.at[b, pl.ds(pos0, _PPW)], idx_v.at[b])

    def start_gathers(i):
        g = i % _NGRP
        return [
            pltpu.async_copy(
                table.at[idx_v.at[b, pl.ds(i * _CH, _CH)]], grp[g][b], gsems[g])
            for b in range(_B)
        ]

    def start_pe(i):
        return pltpu.async_copy(
            pe.at[pl.ds((pos0 + i * _CH) // 2, _CH // 2)],
            pes[i & 1], psems[i & 1])

    def start_stores(i):
        g = i % _NGRP
        return [
            pltpu.async_copy(
                grp[g][b], out.at[b, pl.ds(pos0 + i * _CH, _CH)], ssems[g])
            for b in range(_B)
        ]

    gd = {0: start_gathers(0)}
    pd = {0: start_pe(0)}
    sd = {}
    for i in range(_NCHUNK):
        g = i % _NGRP
        if i + 1 < _NCHUNK:
            if i + 1 >= _NGRP:
                for d in sd.pop(i + 1 - _NGRP):
                    d.wait()
            gd[i + 1] = start_gathers(i + 1)
            pd[i + 1] = start_pe(i + 1)
        pd.pop(i).wait()
        for d in gd.pop(i):
            d.wait()
        _compute(grp[g], pes[i & 1])
        sd[i] = start_stores(i)
    for i in sorted(sd):
        for d in sd[i]:
            d.wait()


def _embed(seqs, table, pe):
    k = pl.kernel(
        _body,
        out_type=jax.ShapeDtypeStruct((_B, _S, _D), jnp.float32),
        mesh=plsc.VectorSubcoreMesh(core_axis_name="c", subcore_axis_name="s"),
        scratch_types=[
            pltpu.VMEM((_B, _PPW), jnp.int32),
            pltpu.VMEM((_CH // 2, 2, _D), jnp.bfloat16),
            pltpu.VMEM((_CH // 2, 2, _D), jnp.bfloat16),
        ]
        + [pltpu.VMEM((_CH, _D), jnp.float32) for _ in range(_NGRP * _B)]
        + [pltpu.SemaphoreType.DMA for _ in range(2 * _NGRP + 2)],
    )
    return k(table, seqs, pe)


def kernel(seqs, embed_weight):
    pe = jnp.asarray(_PE).astype(jnp.bfloat16).reshape(_S // 2, 2, _D)
    return jax.jit(_embed)(seqs, embed_weight, pe)


# speedup vs baseline: 2.9992x; 2.9992x over previous
import math

import numpy as np
import jax
import jax.numpy as jnp
from jax import lax
from jax.experimental import pallas as pl
from jax.experimental.pallas import tpu as pltpu
from jax.experimental.pallas import tpu_sc as plsc

_D = 1024
_B = 4
_S = 4096
_NC = 2
_NS = 16
_NW = _NC * _NS
_PPW = _S // _NW
_CH = 8
_NCHUNK = _PPW // _CH
_SCALE = math.sqrt(_D)
_LANES = 16
_VPR = _D // _LANES
_NGRP = 3


def _pe_rows(pos: np.ndarray) -> np.ndarray:
    omega = np.power(10000.0, -2.0 * np.arange(_D // 2, dtype=np.float64) / _D)
    angle = pos.astype(np.float64)[:, None] * omega[None, :]
    rows = np.empty((pos.shape[0], _D), dtype=np.float64)
    rows[:, 0::2] = np.sin(angle)
    rows[:, 1::2] = np.cos(angle)
    return rows


def _pe_consts() -> tuple[np.ndarray, np.ndarray]:
    omega = np.power(10000.0, -2.0 * np.arange(_D // 2, dtype=np.float64) / _D)
    cdsd = np.empty((2, _D), dtype=np.float64)
    cdsd[0, 0::2] = cdsd[0, 1::2] = np.cos(omega)
    cdsd[1, 0::2] = np.sin(omega)
    cdsd[1, 1::2] = -np.sin(omega)
    pos = np.arange(_NW, dtype=np.float64) * _PPW - 1.0
    p = _pe_rows(pos)
    seeds = np.empty((2 * _NW, _D), dtype=np.float64)
    seeds[0::2] = p
    seeds[1::2, 0::2] = p[:, 1::2]
    seeds[1::2, 1::2] = p[:, 0::2]
    return cdsd.astype(np.float32), seeds.astype(np.float32)


_CDSD, _SEEDS = _pe_consts()


def _compute(bufs, carry, cdsd):
    @plsc.parallel_loop(0, _VPR, unroll=2)
    def _vecs(j):
        sl = pl.ds(j * _LANES, _LANES)
        cd = cdsd[0, sl]
        sd = cdsd[1, sl]
        p = carry[0, sl]
        q = carry[1, sl]
        for r in range(_CH):
            p, q = p * cd + q * sd, q * cd - p * sd
            for buf in bufs:
                buf[r, sl] = buf[r, sl] * _SCALE + p
        carry[0, sl] = p
        carry[1, sl] = q


def _body(table, seqs, cdsd_hbm, seeds_hbm, out, idx_v, cdsd_v, carry, *rest):
    grp = [[rest[g * _B + b] for b in range(_B)] for g in range(_NGRP)]
    gsems = rest[_NGRP * _B:_NGRP * _B + _NGRP]
    ssems = rest[_NGRP * _B + _NGRP:]

    wid = lax.axis_index("s") * _NC + lax.axis_index("c")
    pos0 = wid * _PPW

    for b in range(_B):
        pltpu.sync_copy(seqs.at[b, pl.ds(pos0, _PPW)], idx_v.at[b])
    pltpu.sync_copy(cdsd_hbm, cdsd_v)
    pltpu.sync_copy(seeds_hbm.at[pl.ds(2 * wid, 2)], carry)

    def start_gathers(i):
        g = i % _NGRP
        return [
            pltpu.async_copy(
                table.at[idx_v.at[b, pl.ds(i * _CH, _CH)]], grp[g][b], gsems[g])
            for b in range(_B)
        ]

    def start_stores(i):
        g = i % _NGRP
        return [
            pltpu.async_copy(
                grp[g][b], out.at[b, pl.ds(pos0 + i * _CH, _CH)], ssems[g])
            for b in range(_B)
        ]

    gd = {0: start_gathers(0)}
    sd = {}
    for i in range(_NCHUNK):
        g = i % _NGRP
        if i + 1 < _NCHUNK:
            if i + 1 >= _NGRP:
                for d in sd.pop(i + 1 - _NGRP):
                    d.wait()
            gd[i + 1] = start_gathers(i + 1)
        for d in gd.pop(i):
            d.wait()
        _compute(grp[g], carry, cdsd_v)
        sd[i] = start_stores(i)
    for i in sorted(sd):
        for d in sd[i]:
            d.wait()


def _embed(seqs, table, cdsd, seeds):
    k = pl.kernel(
        _body,
        out_type=jax.ShapeDtypeStruct((_B, _S, _D), jnp.float32),
        mesh=plsc.VectorSubcoreMesh(core_axis_name="c", subcore_axis_name="s"),
        scratch_types=[
            pltpu.VMEM((_B, _PPW), jnp.int32),
            pltpu.VMEM((2, _D), jnp.float32),
            pltpu.VMEM((2, _D), jnp.float32),
        ]
        + [pltpu.VMEM((_CH, _D), jnp.float32) for _ in range(_NGRP * _B)]
        + [pltpu.SemaphoreType.DMA for _ in range(2 * _NGRP)],
    )
    return k(table, seqs, cdsd, seeds)


def kernel(seqs, embed_weight):
    cdsd = jnp.asarray(_CDSD)
    seeds = jnp.asarray(_SEEDS)
    return jax.jit(_embed)(seqs, embed_weight, cdsd, seeds)
